# baseline (speedup 1.0000x reference)
import functools

import jax
import jax.numpy as jnp
from jax import lax
from jax.experimental import pallas as pl
from jax.experimental.pallas import tpu as pltpu
from jax.experimental.pallas import tpu_sc as plsc

V = 1000
D = 1000
DP = 1024
N = 51200
NC, NS, L = 2, 16, 16
NW = NC * NS
PER_W = N // NW
CH = 40
NCH = PER_W // CH
NB = 2
BR = 512
GRID = N // BR


def _lse_body(table_ref, out_ref):
    t = table_ref[...]
    m = jnp.max(t, axis=1)
    s = jnp.sum(jnp.exp(t - m[:, None]), axis=1)
    out_ref[...] = (m + jnp.log(s))[:, None]


def _compute_lse(table):
    return pl.pallas_call(
        _lse_body,
        out_shape=jax.ShapeDtypeStruct((V, 1), jnp.float32),
    )(table)


def _cnt_body(idx_ref, out_ref):
    i = pl.program_id(0)

    @pl.when(i == 0)
    def _():
        out_ref[...] = jnp.zeros_like(out_ref)

    iv = idx_ref[...]
    oh = (iv == lax.broadcasted_iota(jnp.int32, (1, DP), 1)).astype(jnp.float32)
    out_ref[...] += jnp.sum(oh, axis=0, keepdims=True)


def _count_idx(idx_col):
    return pl.pallas_call(
        _cnt_body,
        grid=(GRID,),
        in_specs=[pl.BlockSpec((BR, 1), lambda i: (i, 0))],
        out_specs=pl.BlockSpec((1, DP), lambda i: (0, 0)),
        out_shape=jax.ShapeDtypeStruct((1, DP), jnp.float32),
    )(idx_col)


def _fmt_body(rows_ref, targ_ref, out_ref, part_ref):
    rows = rows_ref[...]
    tv = targ_ref[...]
    out_ref[...] = rows[:, :D]
    oh = tv == lax.broadcasted_iota(jnp.int32, (1, DP), 1)
    part_ref[...] = jnp.full(
        (1, 1, 1), jnp.sum(jnp.where(oh, rows, 0.0)), jnp.float32
    )


def _depad_and_pick(rows_pad, targ_col):
    return pl.pallas_call(
        _fmt_body,
        grid=(GRID,),
        in_specs=[
            pl.BlockSpec((BR, DP), lambda i: (i, 0)),
            pl.BlockSpec((BR, 1), lambda i: (i, 0)),
        ],
        out_specs=[
            pl.BlockSpec((BR, D), lambda i: (i, 0)),
            pl.BlockSpec((1, 1, 1), lambda i: (i, 0, 0)),
        ],
        out_shape=[
            jax.ShapeDtypeStruct((N, D), jnp.float32),
            jax.ShapeDtypeStruct((GRID, 1, 1), jnp.float32),
        ],
    )(rows_pad, targ_col)


def _fin_body(cnt_ref, lse_ref, part_ref, out_ref):
    s1 = (cnt_ref[:, :V] @ lse_ref[...])[0, 0]
    s2 = jnp.sum(part_ref[...])
    out_ref[...] = jnp.full((1, 1), (s1 - s2) / N, jnp.float32)


def _finalize(cnt, lse, parts):
    return pl.pallas_call(
        _fin_body,
        out_shape=jax.ShapeDtypeStruct((1, 1), jnp.float32),
    )(cnt, lse, parts)


def _gather_body(table_hbm, idx_hbm, out_hbm,
                 idx_v, rows_v, gsem0, gsem1, ssem0, ssem1):
    gsems = (gsem0, gsem1)
    ssems = (ssem0, ssem1)
    wid = lax.axis_index("s") * NC + lax.axis_index("c")
    base = wid * PER_W
    pltpu.sync_copy(idx_hbm.at[pl.ds(base, PER_W)], idx_v)

    for b in range(NB):
        pltpu.async_copy(
            table_hbm.at[idx_v.at[pl.ds(b * CH, CH)]], rows_v.at[b], gsems[b]
        )

    def outer(o, _):
        for b in range(NB):
            c = o * NB + b
            off = c * CH
            buf = rows_v.at[b]
            pltpu.make_async_copy(
                table_hbm.at[idx_v.at[pl.ds(off, CH)]], buf, gsems[b]
            ).wait()
            sc_desc = pltpu.async_copy(
                buf, out_hbm.at[pl.ds(base + off, CH)], ssems[b]
            )
            sc_desc.wait()
            nc = c + NB

            @pl.when(nc < NCH)
            def _():
                pltpu.async_copy(
                    table_hbm.at[idx_v.at[pl.ds(nc * CH, CH)]], buf, gsems[b]
                )
        return 0

    lax.fori_loop(0, NCH // NB, outer, 0)


@functools.cache
def _gather_fn():
    mesh = plsc.VectorSubcoreMesh(
        core_axis_name="c", subcore_axis_name="s",
        num_cores=NC, num_subcores=NS,
    )
    return pl.kernel(
        _gather_body,
        out_type=jax.ShapeDtypeStruct((N, DP), jnp.float32),
        mesh=mesh,
        compiler_params=pltpu.CompilerParams(use_tc_tiling_on_sc=True),
        scratch_types=[
            pltpu.VMEM((PER_W,), jnp.int32),
            pltpu.VMEM((NB, CH, DP), jnp.float32),
            pltpu.SemaphoreType.DMA,
            pltpu.SemaphoreType.DMA,
            pltpu.SemaphoreType.DMA,
            pltpu.SemaphoreType.DMA,
        ],
    )


def kernel(idxs, targs, table):
    idx_flat = idxs.reshape(-1)
    targ_flat = targs.reshape(-1)
    table_pad = jnp.pad(table, ((0, 0), (0, DP - D)))
    lse = _compute_lse(table)
    cnt = _count_idx(idx_flat[:, None])
    rows_pad = _gather_fn()(table_pad, idx_flat)
    lgits2, parts = _depad_and_pick(rows_pad, targ_flat[:, None])
    loss = _finalize(cnt, lse, parts)[0, 0]
    return (lgits2, loss)


# speedup vs baseline: 1.1346x; 1.1346x over previous
import functools

import jax
import jax.numpy as jnp
from jax import lax
from jax.experimental import pallas as pl
from jax.experimental.pallas import tpu as pltpu
from jax.experimental.pallas import tpu_sc as plsc

V = 1000
D = 1000
DP = 1024
N = 51200
NC, NS, L = 2, 16, 16
NW = NC * NS
PER_W = N // NW
CH = 32
NCH = PER_W // CH
NB = 2
BR = 512
GRID = N // BR


def _lse_body(table_ref, out_ref):
    t = table_ref[...]
    m = jnp.max(t, axis=1)
    s = jnp.sum(jnp.exp(t - m[:, None]), axis=1)
    out_ref[...] = (m + jnp.log(s))[:, None]


def _compute_lse(table):
    return pl.pallas_call(
        _lse_body,
        out_shape=jax.ShapeDtypeStruct((V, 1), jnp.float32),
    )(table)


def _cnt_body(idx_ref, out_ref):
    i = pl.program_id(0)

    @pl.when(i == 0)
    def _():
        out_ref[...] = jnp.zeros_like(out_ref)

    iv = idx_ref[...]
    oh = (iv == lax.broadcasted_iota(jnp.int32, (1, DP), 1)).astype(jnp.float32)
    out_ref[...] += jnp.sum(oh, axis=0, keepdims=True)


def _count_idx(idx_col):
    return pl.pallas_call(
        _cnt_body,
        grid=(GRID,),
        in_specs=[pl.BlockSpec((BR, 1), lambda i: (i, 0))],
        out_specs=pl.BlockSpec((1, DP), lambda i: (0, 0)),
        out_shape=jax.ShapeDtypeStruct((1, DP), jnp.float32),
    )(idx_col)


def _pick_body(rows_ref, targ_ref, part_ref):
    rows = rows_ref[...]
    tv = targ_ref[...]
    oh = tv == lax.broadcasted_iota(jnp.int32, (1, DP), 1)
    part_ref[...] = jnp.full(
        (1, 1, 1), jnp.sum(jnp.where(oh, rows, 0.0)), jnp.float32
    )


def _pick(rows_pad, targ_col):
    return pl.pallas_call(
        _pick_body,
        grid=(GRID,),
        in_specs=[
            pl.BlockSpec((BR, DP), lambda i: (i, 0)),
            pl.BlockSpec((BR, 1), lambda i: (i, 0)),
        ],
        out_specs=pl.BlockSpec((1, 1, 1), lambda i: (i, 0, 0)),
        out_shape=jax.ShapeDtypeStruct((GRID, 1, 1), jnp.float32),
    )(rows_pad, targ_col)


def _fin_body(cnt_ref, lse_ref, part_ref, out_ref):
    s1 = (cnt_ref[:, :V] @ lse_ref[...])[0, 0]
    s2 = jnp.sum(part_ref[...])
    out_ref[...] = jnp.full((1, 1), (s1 - s2) / N, jnp.float32)


def _finalize(cnt, lse, parts):
    return pl.pallas_call(
        _fin_body,
        out_shape=jax.ShapeDtypeStruct((1, 1), jnp.float32),
    )(cnt, lse, parts)


def _gather_body(table_hbm, idx_hbm, out_hbm,
                 idx_v, rows_v, gsem0, gsem1, ssem0, ssem1):
    gsems = (gsem0, gsem1)
    ssems = (ssem0, ssem1)
    wid = lax.axis_index("s") * NC + lax.axis_index("c")
    base = wid * PER_W
    pltpu.sync_copy(idx_hbm.at[pl.ds(base, PER_W)], idx_v)

    for b in range(NB):
        pltpu.async_copy(
            table_hbm.at[idx_v.at[pl.ds(b * CH, CH)]], rows_v.at[b], gsems[b]
        )

    def outer(o, _):
        for b in range(NB):
            c = o * NB + b
            off = c * CH
            buf = rows_v.at[b]
            pltpu.make_async_copy(
                table_hbm.at[idx_v.at[pl.ds(off, CH)]], buf, gsems[b]
            ).wait()
            sc_desc = pltpu.async_copy(
                buf, out_hbm.at[pl.ds(base + off, CH)], ssems[b]
            )
            sc_desc.wait()
            nc = c + NB

            @pl.when(nc < NCH)
            def _():
                pltpu.async_copy(
                    table_hbm.at[idx_v.at[pl.ds(nc * CH, CH)]], buf, gsems[b]
                )
        return 0

    lax.fori_loop(0, NCH // NB, outer, 0)


@functools.cache
def _gather_fn():
    mesh = plsc.VectorSubcoreMesh(
        core_axis_name="c", subcore_axis_name="s",
        num_cores=NC, num_subcores=NS,
    )
    return pl.kernel(
        _gather_body,
        out_type=jax.ShapeDtypeStruct((N, DP), jnp.float32),
        mesh=mesh,
        compiler_params=pltpu.CompilerParams(use_tc_tiling_on_sc=True),
        scratch_types=[
            pltpu.VMEM((PER_W,), jnp.int32),
            pltpu.VMEM((NB, CH, DP), jnp.float32),
            pltpu.SemaphoreType.DMA,
            pltpu.SemaphoreType.DMA,
            pltpu.SemaphoreType.DMA,
            pltpu.SemaphoreType.DMA,
        ],
    )


def kernel(idxs, targs, table):
    idx_flat = idxs.reshape(-1)
    targ_flat = targs.reshape(-1)
    table_pad = jnp.pad(table, ((0, 0), (0, DP - D)))
    lse = _compute_lse(table)
    cnt = _count_idx(idx_flat[:, None])
    rows_pad = _gather_fn()(table_pad, idx_flat)
    parts = _pick(rows_pad, targ_flat[:, None])
    lgits2 = rows_pad[:, :D]
    loss = _finalize(cnt, lse, parts)[0, 0]
    return (lgits2, loss)


# speedup vs baseline: 1.3115x; 1.1559x over previous
import functools

import jax
import jax.numpy as jnp
from jax import lax
from jax.experimental import pallas as pl
from jax.experimental.pallas import tpu as pltpu
from jax.experimental.pallas import tpu_sc as plsc

V = 1000
D = 1000
DP = 1024
N = 51200
NC, NS, L = 2, 16, 16
NW = NC * NS
PER_W = N // NW
CH = 32
NCH = PER_W // CH
NB = 2
BR = 512
GRID = N // BR


def _lse_body(table_ref, out_ref):
    t = table_ref[...]
    m = jnp.max(t, axis=1)
    s = jnp.sum(jnp.exp(t - m[:, None]), axis=1)
    out_ref[...] = (m + jnp.log(s))[:, None]


def _compute_lse(table):
    return pl.pallas_call(
        _lse_body,
        out_shape=jax.ShapeDtypeStruct((V, 1), jnp.float32),
    )(table)


def _pick_body(rows_ref, targ_ref, part_ref):
    rows = rows_ref[...]
    tv = targ_ref[...]
    oh = tv == lax.broadcasted_iota(jnp.int32, (1, DP), 1)
    part_ref[...] = jnp.full(
        (1, 1, 1), jnp.sum(jnp.where(oh, rows, 0.0)), jnp.float32
    )


def _pick(rows_pad, targ_col):
    return pl.pallas_call(
        _pick_body,
        grid=(GRID,),
        in_specs=[
            pl.BlockSpec((BR, DP), lambda i: (i, 0)),
            pl.BlockSpec((BR, 1), lambda i: (i, 0)),
        ],
        out_specs=pl.BlockSpec((1, 1, 1), lambda i: (i, 0, 0)),
        out_shape=jax.ShapeDtypeStruct((GRID, 1, 1), jnp.float32),
    )(rows_pad, targ_col)


def _fin_body(cnt_ref, lse_ref, part_ref, out_ref):
    c2 = jnp.sum(cnt_ref[...], axis=0)
    s1 = (c2[:, :V] @ lse_ref[...])[0, 0]
    s2 = jnp.sum(part_ref[...])
    out_ref[...] = jnp.full((1, 1), (s1 - s2) / N, jnp.float32)


def _finalize(cnt, lse, parts):
    return pl.pallas_call(
        _fin_body,
        out_shape=jax.ShapeDtypeStruct((1, 1), jnp.float32),
    )(cnt, lse, parts)


def _gather_body(table_hbm, idx_hbm, out_hbm, cnt_hbm,
                 idx_v, rows_v, cnt_v, gsem0, gsem1, ssem0, ssem1):
    gsems = (gsem0, gsem1)
    ssems = (ssem0, ssem1)
    wid = lax.axis_index("s") * NC + lax.axis_index("c")
    base = wid * PER_W
    pltpu.sync_copy(idx_hbm.at[pl.ds(base, PER_W)], idx_v)

    for b in range(NB):
        pltpu.async_copy(
            table_hbm.at[idx_v.at[pl.ds(b * CH, CH)]], rows_v.at[b], gsems[b]
        )

    def zero(g, _):
        cnt_v[pl.ds(g * L, L)] = jnp.zeros((L,), jnp.float32)
        return 0

    lax.fori_loop(0, DP // L, zero, 0)

    ones = jnp.ones((L,), jnp.float32)

    def hist(g, _):
        iv = idx_v[pl.ds(g * L, L)]
        plsc.addupdate_scatter(cnt_v, [iv], ones)
        return 0

    lax.fori_loop(0, PER_W // L, hist, 0)

    def outer(o, _):
        for b in range(NB):
            c = o * NB + b
            off = c * CH
            buf = rows_v.at[b]
            pltpu.make_async_copy(
                table_hbm.at[idx_v.at[pl.ds(off, CH)]], buf, gsems[b]
            ).wait()
            sc_desc = pltpu.async_copy(
                buf, out_hbm.at[pl.ds(base + off, CH)], ssems[b]
            )
            sc_desc.wait()
            nc = c + NB

            @pl.when(nc < NCH)
            def _():
                pltpu.async_copy(
                    table_hbm.at[idx_v.at[pl.ds(nc * CH, CH)]], buf, gsems[b]
                )
        return 0

    lax.fori_loop(0, NCH // NB, outer, 0)
    pltpu.sync_copy(cnt_v, cnt_hbm.at[wid, 0])


@functools.cache
def _gather_fn():
    mesh = plsc.VectorSubcoreMesh(
        core_axis_name="c", subcore_axis_name="s",
        num_cores=NC, num_subcores=NS,
    )
    return pl.kernel(
        _gather_body,
        out_type=(
            jax.ShapeDtypeStruct((N, DP), jnp.float32),
            jax.ShapeDtypeStruct((NW, 1, DP), jnp.float32),
        ),
        mesh=mesh,
        compiler_params=pltpu.CompilerParams(
            use_tc_tiling_on_sc=True, needs_layout_passes=False
        ),
        scratch_types=[
            pltpu.VMEM((PER_W,), jnp.int32),
            pltpu.VMEM((NB, CH, DP), jnp.float32),
            pltpu.VMEM((DP,), jnp.float32),
            pltpu.SemaphoreType.DMA,
            pltpu.SemaphoreType.DMA,
            pltpu.SemaphoreType.DMA,
            pltpu.SemaphoreType.DMA,
        ],
    )


def kernel(idxs, targs, table):
    idx_flat = idxs.reshape(-1)
    targ_flat = targs.reshape(-1)
    table_pad = jnp.pad(table, ((0, 0), (0, DP - D)))
    lse = _compute_lse(table)
    rows_pad, cnt = _gather_fn()(table_pad, idx_flat)
    parts = _pick(rows_pad, targ_flat[:, None])
    lgits2 = rows_pad[:, :D]
    loss = _finalize(cnt, lse, parts)[0, 0]
    return (lgits2, loss)


# speedup vs baseline: 1.6556x; 1.2624x over previous
import functools

import jax
import jax.numpy as jnp
from jax import lax
from jax.experimental import pallas as pl
from jax.experimental.pallas import tpu as pltpu
from jax.experimental.pallas import tpu_sc as plsc

V = 1000
D = 1000
DP = 1024
N = 51200
NC, NS, L = 2, 16, 16
NW = NC * NS
PER_W = N // NW
CH = 32
NCH = PER_W // CH
NB = 2
TFLAT = V * D + 8


def _lse_body(table_ref, out_ref):
    t = table_ref[...]
    m = jnp.max(t, axis=1)
    s = jnp.sum(jnp.exp(t - m[:, None]), axis=1)
    out_ref[...] = (m + jnp.log(s))[:, None]


def _compute_lse(table):
    return pl.pallas_call(
        _lse_body,
        out_shape=jax.ShapeDtypeStruct((V, 1), jnp.float32),
    )(table)


def _fin_body(cnt_ref, lse_ref, out_ref):
    c2 = jnp.sum(cnt_ref[...], axis=0)
    s1 = (c2[:, :V] @ lse_ref[...])[0, 0]
    s2 = jnp.sum(c2[:, V + 8:])
    out_ref[...] = jnp.full((1, 1), (s1 - s2) / N, jnp.float32)


def _finalize(cnt, lse):
    return pl.pallas_call(
        _fin_body,
        out_shape=jax.ShapeDtypeStruct((1, 1), jnp.float32),
    )(cnt, lse)


def _gather_body(table_hbm, idx_hbm, tflat_hbm, fidx_hbm, out_hbm, cnt_hbm,
                 idx_v, rows_v, cnt_v, fidx_v, picked_v,
                 gsem0, gsem1, ssem0, ssem1, psem):
    gsems = (gsem0, gsem1)
    ssems = (ssem0, ssem1)
    wid = lax.axis_index("s") * NC + lax.axis_index("c")
    base = wid * PER_W
    pltpu.sync_copy(idx_hbm.at[pl.ds(base, PER_W)], idx_v)
    pltpu.sync_copy(fidx_hbm.at[pl.ds(base, PER_W)], fidx_v)
    pick_desc = pltpu.async_copy(tflat_hbm.at[fidx_v], picked_v, psem)

    for b in range(NB):
        pltpu.async_copy(
            table_hbm.at[idx_v.at[pl.ds(b * CH, CH)]], rows_v.at[b], gsems[b]
        )

    def zero(g, _):
        cnt_v[pl.ds(g * L, L)] = jnp.zeros((L,), jnp.float32)
        return 0

    lax.fori_loop(0, DP // L, zero, 0)

    ones = jnp.ones((L,), jnp.float32)

    def hist(g, _):
        iv = idx_v[pl.ds(g * L, L)]
        plsc.addupdate_scatter(cnt_v, [iv], ones)
        return 0

    lax.fori_loop(0, PER_W // L, hist, 0)

    def outer(o, _):
        for b in range(NB):
            c = o * NB + b
            off = c * CH
            buf = rows_v.at[b]
            pltpu.make_async_copy(
                table_hbm.at[idx_v.at[pl.ds(off, CH)]], buf, gsems[b]
            ).wait()
            sc_desc = pltpu.async_copy(
                buf, out_hbm.at[pl.ds(base + off, CH)], ssems[b]
            )
            sc_desc.wait()
            nc = c + NB

            @pl.when(nc < NCH)
            def _():
                pltpu.async_copy(
                    table_hbm.at[idx_v.at[pl.ds(nc * CH, CH)]], buf, gsems[b]
                )
        return 0

    lax.fori_loop(0, NCH // NB, outer, 0)

    pick_desc.wait()

    def red(g, acc):
        return acc + picked_v[pl.ds(g * L, L)]

    acc = lax.fori_loop(0, PER_W // L, red, jnp.zeros((L,), jnp.float32))
    cnt_v[pl.ds(V + 8, L)] = acc
    pltpu.sync_copy(cnt_v, cnt_hbm.at[wid, 0])


@functools.cache
def _gather_fn():
    mesh = plsc.VectorSubcoreMesh(
        core_axis_name="c", subcore_axis_name="s",
        num_cores=NC, num_subcores=NS,
    )
    return pl.kernel(
        _gather_body,
        out_type=(
            jax.ShapeDtypeStruct((N, DP), jnp.float32),
            jax.ShapeDtypeStruct((NW, 1, DP), jnp.float32),
        ),
        mesh=mesh,
        compiler_params=pltpu.CompilerParams(
            use_tc_tiling_on_sc=True, needs_layout_passes=False
        ),
        scratch_types=[
            pltpu.VMEM((PER_W,), jnp.int32),
            pltpu.VMEM((NB, CH, DP), jnp.float32),
            pltpu.VMEM((DP,), jnp.float32),
            pltpu.VMEM((PER_W,), jnp.int32),
            pltpu.VMEM((PER_W,), jnp.float32),
            pltpu.SemaphoreType.DMA,
            pltpu.SemaphoreType.DMA,
            pltpu.SemaphoreType.DMA,
            pltpu.SemaphoreType.DMA,
            pltpu.SemaphoreType.DMA,
        ],
    )


def kernel(idxs, targs, table):
    idx_flat = idxs.reshape(-1)
    targ_flat = targs.reshape(-1)
    fidx = idx_flat * D + targ_flat
    table_pad = jnp.pad(table, ((0, 0), (0, DP - D)))
    tflat = jnp.concatenate(
        [table.reshape(-1), jnp.zeros((TFLAT - V * D,), jnp.float32)]
    )
    lse = _compute_lse(table)
    rows_pad, cnt = _gather_fn()(table_pad, idx_flat, tflat, fidx)
    lgits2 = rows_pad[:, :D]
    loss = _finalize(cnt, lse)[0, 0]
    return (lgits2, loss)
